# Initial kernel scaffold; baseline (speedup 1.0000x reference)
import jax
import jax.numpy as jnp
from jax.experimental import pallas as pl


def kernel(x, grid):
    raise NotImplementedError("write your pallas kernel here")


# speedup vs baseline: 3.8340x; 3.8340x over previous
import functools

import jax
import jax.numpy as jnp
from jax import lax
from jax.experimental import pallas as pl
from jax.experimental.pallas import tpu as pltpu
from jax.experimental.pallas import tpu_sc as plsc

N_POINTS = 262144
N_LEVELS = 8
N_FEATS = 2
LOD_RES = (16, 32, 64, 128, 256, 512, 1024, 2048)
HASHMAP_SIZE = 2 ** 19
P1 = 2654435761
P2 = 805459861

_LEVEL_SIZES = tuple(int(min((r + 1) ** 3, HASHMAP_SIZE)) for r in LOD_RES)
_LEVEL_OFFS = []
_acc = 0
for _s in _LEVEL_SIZES:
    _LEVEL_OFFS.append(_acc)
    _acc += _s
_LEVEL_OFFS = tuple(_LEVEL_OFFS)
N_TABLE_ROWS = _acc

NW = 32
PPT = N_POINTS // NW
C = 256
NCHUNK = PPT // C
G = C // 16
NSTREAM = C * 128 // 128
FIRE = 8


def _corner_hashes(px, py, pz, res):
    posx = px * float(res)
    posy = py * float(res)
    posz = pz * float(res)
    ix = posx.astype(jnp.int32)
    iy = posy.astype(jnp.int32)
    iz = posz.astype(jnp.int32)
    a0 = ix.astype(jnp.uint32)
    b0 = iy.astype(jnp.uint32) * jnp.uint32(P1)
    c0 = iz.astype(jnp.uint32) * jnp.uint32(P2)
    a1 = a0 + jnp.uint32(1)
    b1 = b0 + jnp.uint32(P1)
    c1 = c0 + jnp.uint32(P2)
    ab = (a0 ^ b0, a0 ^ b1, a1 ^ b0, a1 ^ b1)
    hs = []
    for c in range(8):
        dx, dy, dz = (c >> 2) & 1, (c >> 1) & 1, c & 1
        hs.append(ab[2 * dx + dy] ^ (c1 if dz else c0))
    return hs


def _fracs(px, py, pz, res):
    posx = px * float(res)
    posy = py * float(res)
    posz = pz * float(res)
    fx = posx - posx.astype(jnp.int32).astype(jnp.float32)
    fy = posy - posy.astype(jnp.int32).astype(jnp.float32)
    fz = posz - posz.astype(jnp.int32).astype(jnp.float32)
    return fx, fy, fz


def _body(xt, table, out, xb, idx, dst, ob, sem):
    wid = lax.axis_index("s") * 2 + lax.axis_index("c")
    base = wid * PPT

    iota = lax.iota(jnp.int32, 16)

    @pl.loop(0, NCHUNK)
    def _chunk(n):
        start = base + n * C
        for d in range(3):
            pltpu.sync_copy(xt.at[pl.ds(d * N_POINTS + start, C)],
                            xb.at[pl.ds(d * C, C)])

        @pl.loop(0, G)
        def _gen(g):
            px = jnp.clip(xb[pl.ds(0 * C + g * 16, 16)], 1e-6, 1.0 - 1e-6)
            py = jnp.clip(xb[pl.ds(1 * C + g * 16, 16)], 1e-6, 1.0 - 1e-6)
            pz = jnp.clip(xb[pl.ds(2 * C + g * 16, 16)], 1e-6, 1.0 - 1e-6)
            for lvl, res in enumerate(LOD_RES):
                size = _LEVEL_SIZES[lvl]
                off = _LEVEL_OFFS[lvl]
                hs = _corner_hashes(px, py, pz, res)
                rb = (g * 8 + lvl) * 256
                for c in range(8):
                    if size & (size - 1) == 0:
                        hid = hs[c] & jnp.uint32(size - 1)
                    else:
                        hid = hs[c] % jnp.uint32(size)
                    w0 = (hid.astype(jnp.int32) + jnp.int32(off)) * 2
                    idx[pl.ds(rb + 16 * c, 16)] = w0
                    idx[pl.ds(rb + 128 + 16 * c, 16)] = w0 + 1

        @pl.loop(0, NSTREAM, step=FIRE)
        def _stream(j):
            descs = []
            for k in range(FIRE):
                descs.append(
                    pltpu.async_copy(
                        table.at[idx.at[pl.ds((j + k) * 128, 128)]],
                        dst.at[pl.ds((j + k) * 128, 128)],
                        sem,
                    )
                )
            for dsc in descs:
                dsc.wait()

        @pl.loop(0, G)
        def _interp(g):
            px = jnp.clip(xb[pl.ds(0 * C + g * 16, 16)], 1e-6, 1.0 - 1e-6)
            py = jnp.clip(xb[pl.ds(1 * C + g * 16, 16)], 1e-6, 1.0 - 1e-6)
            pz = jnp.clip(xb[pl.ds(2 * C + g * 16, 16)], 1e-6, 1.0 - 1e-6)
            obase = g * 256 + iota * 16
            for lvl, res in enumerate(LOD_RES):
                fx, fy, fz = _fracs(px, py, pz, res)
                wx = (1.0 - fx, fx)
                wy = (1.0 - fy, fy)
                wz = (1.0 - fz, fz)
                wxy = (wx[0] * wy[0], wx[0] * wy[1], wx[1] * wy[0], wx[1] * wy[1])
                rb = (g * 8 + lvl) * 256
                acc0 = jnp.zeros((16,), jnp.float32)
                acc1 = jnp.zeros((16,), jnp.float32)
                for c in range(8):
                    dx, dy, dz = (c >> 2) & 1, (c >> 1) & 1, c & 1
                    w = wxy[2 * dx + dy] * wz[dz]
                    f0 = dst[pl.ds(rb + 16 * c, 16)]
                    f1 = dst[pl.ds(rb + 128 + 16 * c, 16)]
                    acc0 = acc0 + f0 * w
                    acc1 = acc1 + f1 * w
                plsc.store_scatter(ob, [obase + (2 * lvl)], acc0)
                plsc.store_scatter(ob, [obase + (2 * lvl + 1)], acc1)

        pltpu.sync_copy(ob, out.at[pl.ds(start * 16, C * 16)])


@functools.cache
def _lotd():
    return pl.kernel(
        _body,
        out_type=jax.ShapeDtypeStruct((N_POINTS * N_LEVELS * N_FEATS,), jnp.float32),
        mesh=plsc.VectorSubcoreMesh(core_axis_name="c", subcore_axis_name="s"),
        compiler_params=pltpu.CompilerParams(needs_layout_passes=False),
        scratch_types=[
            pltpu.VMEM((3 * C,), jnp.float32),
            pltpu.VMEM((NSTREAM * 128,), jnp.int32),
            pltpu.VMEM((NSTREAM * 128,), jnp.float32),
            pltpu.VMEM((C * 16,), jnp.float32),
            pltpu.SemaphoreType.DMA,
        ],
    )


@jax.jit
def kernel(x, grid):
    xt = x.T.reshape(-1)
    flat = _lotd()(xt, grid)
    return flat.reshape(N_POINTS, N_LEVELS * N_FEATS)
